# Initial kernel scaffold; baseline (speedup 1.0000x reference)
import jax
import jax.numpy as jnp
from jax.experimental import pallas as pl


def kernel(x_src, x_dst, edge_index, W, b):
    raise NotImplementedError("write your pallas kernel here")


# speedup vs baseline: 4.4416x; 4.4416x over previous
import functools

import jax
import jax.numpy as jnp
from jax import lax
from jax.experimental import pallas as pl
from jax.experimental.pallas import tpu as pltpu
from jax.experimental.pallas import tpu_sc as plsc

NC = 2
NS = 16
NW = NC * NS
CHUNK = 128
CNT_W = 128
ZROWS = 32


def _sc_aggregate(src_chunks, dst_chunks, x_src, acc_rows, cpw):
  n_nodes, d_feat = x_src.shape
  rows_per_sub = acc_rows // NS
  d_lanes = d_feat // 16

  mesh = plsc.VectorSubcoreMesh(core_axis_name="c", subcore_axis_name="s")

  @functools.partial(
      pl.kernel,
      mesh=mesh,
      out_type=jax.ShapeDtypeStruct((NC, acc_rows, d_feat), jnp.float32),
      scratch_types=[
          pltpu.VMEM((CHUNK,), jnp.int32),
          pltpu.VMEM((CHUNK,), jnp.int32),
          pltpu.VMEM((CHUNK, d_feat), jnp.float32),
          pltpu.VMEM((ZROWS, d_feat), jnp.float32),
          pltpu.VMEM_SHARED((acc_rows, d_feat), jnp.float32),
          pltpu.SemaphoreType.DMA,
      ],
  )
  def sc_kernel(srcc_hbm, dstc_hbm, xsrc_hbm, sum_out,
                src_v, dst_v, rows_v, zrow_v, acc_sh, sem):
    c = lax.axis_index("c")
    s = lax.axis_index("s")
    wid = c * NS + s

    zeros16 = jnp.zeros((16,), jnp.float32)
    for r in range(ZROWS):
      for l in range(d_lanes):
        zrow_v[r, pl.ds(16 * l, 16)] = zeros16

    row0 = s * rows_per_sub
    for t in range(rows_per_sub // ZROWS):
      pltpu.sync_copy(zrow_v, acc_sh.at[pl.ds(row0 + ZROWS * t, ZROWS)])
    plsc.subcore_barrier()

    def chunk_body(j, carry):
      ch = wid * cpw + j
      pltpu.sync_copy(srcc_hbm.at[ch], src_v)
      pltpu.sync_copy(dstc_hbm.at[ch], dst_v)
      pltpu.async_copy(xsrc_hbm.at[src_v], rows_v, sem).wait()
      pltpu.sync_copy(rows_v, acc_sh.at[dst_v], add=True)
      return carry

    lax.fori_loop(0, cpw, chunk_body, 0)
    plsc.subcore_barrier()

    pltpu.sync_copy(acc_sh.at[pl.ds(row0, rows_per_sub)],
                    sum_out.at[c, pl.ds(row0, rows_per_sub)])

  return sc_kernel(src_chunks, dst_chunks, x_src)


def _sc_counts(dst_chunks, acc_rows, cpw):
  rows_per_sub = acc_rows // NS

  mesh = plsc.VectorSubcoreMesh(core_axis_name="c", subcore_axis_name="s")

  @functools.partial(
      pl.kernel,
      mesh=mesh,
      out_type=jax.ShapeDtypeStruct((NC, acc_rows, CNT_W), jnp.float32),
      scratch_types=[
          pltpu.VMEM((CHUNK,), jnp.int32),
          pltpu.VMEM((CHUNK, CNT_W), jnp.float32),
          pltpu.VMEM((ZROWS, CNT_W), jnp.float32),
          pltpu.VMEM_SHARED((acc_rows, CNT_W), jnp.float32),
      ],
  )
  def cnt_kernel(dstc_hbm, cnt_out, dst_v, ones_v, zcnt_v, cnt_sh):
    c = lax.axis_index("c")
    s = lax.axis_index("s")
    wid = c * NS + s

    zeros16 = jnp.zeros((16,), jnp.float32)
    ones16 = jnp.ones((16,), jnp.float32)
    cnt_lanes = CNT_W // 16
    for r in range(ZROWS):
      for l in range(cnt_lanes):
        zcnt_v[r, pl.ds(16 * l, 16)] = zeros16
    for r in range(CHUNK):
      for l in range(cnt_lanes):
        ones_v[r, pl.ds(16 * l, 16)] = ones16

    row0 = s * rows_per_sub
    for t in range(rows_per_sub // ZROWS):
      pltpu.sync_copy(zcnt_v, cnt_sh.at[pl.ds(row0 + ZROWS * t, ZROWS)])
    plsc.subcore_barrier()

    def chunk_body(j, carry):
      ch = wid * cpw + j
      pltpu.sync_copy(dstc_hbm.at[ch], dst_v)
      pltpu.sync_copy(ones_v, cnt_sh.at[dst_v], add=True)
      return carry

    lax.fori_loop(0, cpw, chunk_body, 0)
    plsc.subcore_barrier()

    pltpu.sync_copy(cnt_sh.at[pl.ds(row0, rows_per_sub)],
                    cnt_out.at[c, pl.ds(row0, rows_per_sub)])

  return cnt_kernel(dst_chunks)


def _tc_finalize(x_dst_pad, sums, cnts, w_t, b2, block_rows):
  acc_rows, d_feat = x_dst_pad.shape
  out_ch = w_t.shape[1]

  def tc_kernel(xd_ref, p_ref, c_ref, wt_ref, b_ref, o_ref):
    p = p_ref[0] + p_ref[1]
    cnt = c_ref[0, :, 0:1] + c_ref[1, :, 0:1]
    agg = p / jnp.maximum(cnt, 1.0)
    wt = wt_ref[...]
    h = jnp.dot(xd_ref[...], wt[:d_feat], precision=lax.Precision.HIGHEST)
    h = h + jnp.dot(agg, wt[d_feat:], precision=lax.Precision.HIGHEST)
    o_ref[...] = jnp.maximum(h + b_ref[...], 0.0)

  return pl.pallas_call(
      tc_kernel,
      grid=(acc_rows // block_rows,),
      in_specs=[
          pl.BlockSpec((block_rows, d_feat), lambda i: (i, 0)),
          pl.BlockSpec((NC, block_rows, d_feat), lambda i: (0, i, 0)),
          pl.BlockSpec((NC, block_rows, CNT_W), lambda i: (0, i, 0)),
          pl.BlockSpec((2 * d_feat, out_ch), lambda i: (0, 0)),
          pl.BlockSpec((1, out_ch), lambda i: (0, 0)),
      ],
      out_specs=pl.BlockSpec((block_rows, out_ch), lambda i: (i, 0)),
      out_shape=jax.ShapeDtypeStruct((acc_rows, out_ch), jnp.float32),
  )(x_dst_pad, sums, cnts, w_t, b2)


def kernel(x_src, x_dst, edge_index, W, b):
  n_nodes, d_feat = x_src.shape
  n_edges = edge_index.shape[1]

  src = edge_index[0].astype(jnp.int32)
  dst = edge_index[1].astype(jnp.int32)

  cpw = -(-n_edges // (CHUNK * NW))
  e_pad = cpw * NW * CHUNK
  pad = e_pad - n_edges
  src = jnp.concatenate([src, jnp.zeros((pad,), jnp.int32)])
  dst = jnp.concatenate([dst, jnp.full((pad,), n_nodes, jnp.int32)])
  src_chunks = src.reshape(NW * cpw, CHUNK)
  dst_chunks = dst.reshape(NW * cpw, CHUNK)

  acc_rows = -(-(n_nodes + 1) // (ZROWS * NS)) * (ZROWS * NS)

  sums = _sc_aggregate(src_chunks, dst_chunks, x_src, acc_rows, cpw)
  cnts = _sc_counts(dst_chunks, acc_rows, cpw)

  x_dst_pad = jnp.concatenate(
      [x_dst, jnp.zeros((acc_rows - n_nodes, d_feat), x_dst.dtype)])

  block_rows = 2048 if acc_rows % 2048 == 0 else 1024
  out = _tc_finalize(x_dst_pad, sums, cnts, W.T, b.reshape(1, -1), block_rows)
  return out[:n_nodes]
